# baseline (speedup 1.0000x reference)
import functools

import jax
import jax.numpy as jnp
from jax import lax
from jax.experimental import pallas as pl
from jax.experimental.pallas import tpu as pltpu
from jax.experimental.pallas import tpu_sc as plsc

VOCAB = 100000
HIDDEN = 768
S_LEN = 512
TT_VOCAB = 2
B_SZ = 64
N_TOK = B_SZ * S_LEN
LN_EPS = 1e-12

NW = 32
SP_W = S_LEN // NW
CHUNK = 16
NCH = B_SZ
HS = HIDDEN // 16
NBROWS = TT_VOCAB * SP_W

_mesh = plsc.VectorSubcoreMesh(core_axis_name="c", subcore_axis_name="s")
_cparams = pltpu.CompilerParams(needs_layout_passes=False)


@functools.partial(
    pl.kernel,
    mesh=_mesh,
    out_type=jax.ShapeDtypeStruct((N_TOK, HIDDEN), jnp.float32),
    compiler_params=_cparams,
    scratch_types=[
        pltpu.VMEM((NCH, CHUNK), jnp.int32),
        pltpu.VMEM((NCH, CHUNK), jnp.int32),
        pltpu.VMEM((CHUNK, HIDDEN), jnp.float32),
        pltpu.VMEM((CHUNK, HIDDEN), jnp.float32),
        pltpu.VMEM((CHUNK, HIDDEN), jnp.float32),
        pltpu.VMEM((CHUNK, HIDDEN), jnp.float32),
        pltpu.VMEM((NBROWS, HIDDEN), jnp.float32),
        pltpu.VMEM((TT_VOCAB, HIDDEN), jnp.float32),
        pltpu.VMEM((HIDDEN,), jnp.float32),
        pltpu.VMEM((HIDDEN,), jnp.float32),
        pltpu.VMEM((CHUNK * 16,), jnp.float32),
        pltpu.VMEM((CHUNK * 16,), jnp.float32),
        pltpu.VMEM((CHUNK,), jnp.float32),
        pltpu.VMEM((CHUNK,), jnp.float32),
        pltpu.SemaphoreType.DMA,
        pltpu.SemaphoreType.DMA,
        pltpu.SemaphoreType.DMA,
        pltpu.SemaphoreType.DMA,
    ],
)
def _emb_layernorm(ids_hbm, brow_hbm, word_hbm, tt_hbm, pos_hbm, gam_hbm,
                   bet_hbm, out_hbm, widx_v, brow_v, rows0_v, rows1_v,
                   outb0_v, outb1_v, ttb_v, ttbuf_v, gam_v, bet_v,
                   sums_v, sq_v, a_v, b_v, sem_w0, sem_w1, sem_o0, sem_o1):
    wid = lax.axis_index("s") * 2 + lax.axis_index("c")
    pltpu.sync_copy(ids_hbm.at[wid], widx_v)
    pltpu.sync_copy(brow_hbm.at[wid], brow_v)
    pltpu.sync_copy(gam_hbm, gam_v)
    pltpu.sync_copy(bet_hbm, bet_v)
    pltpu.sync_copy(tt_hbm, ttbuf_v)
    pltpu.sync_copy(pos_hbm.at[pl.ds(wid * SP_W, SP_W)],
                    ttb_v.at[pl.ds(0, SP_W)])
    pltpu.sync_copy(pos_hbm.at[pl.ds(wid * SP_W, SP_W)],
                    ttb_v.at[pl.ds(SP_W, SP_W)])

    @plsc.parallel_loop(0, NBROWS * HS, unroll=8)
    def build(i):
        r = i // HS
        hs = pl.ds((i % HS) * 16, 16)
        ttb_v[r, hs] = ttb_v[r, hs] + ttbuf_v[r // SP_W, hs]

    lane = lax.iota(jnp.int32, 16)
    bufs = ((rows0_v, outb0_v, sem_w0, sem_o0),
            (rows1_v, outb1_v, sem_w1, sem_o1))

    def issue(ci, rows, sw):
        pltpu.async_copy(word_hbm.at[widx_v.at[ci]], rows, sw)

    def compute(ci, rows, outb):
        @plsc.parallel_loop(0, CHUNK)
        def p1(t):
            br = plsc.load_gather(brow_v, [jnp.full((16,), ci, jnp.int32),
                                           jnp.full((16,), t, jnp.int32)])
            z = jnp.zeros((16,), jnp.float32)

            @plsc.parallel_loop(0, HS, unroll=8, carry=(z, z))
            def accs(h, acc):
                acc_s, acc_q = acc
                hs = pl.ds(h * 16, 16)
                bslice = plsc.load_gather(ttb_v, [br, h * 16 + lane])
                v = rows[t, hs] + bslice
                rows[t, hs] = v
                return (acc_s + v, acc_q + v * v)

            acc_s, acc_q = accs
            sums_v[pl.ds(t * 16, 16)] = acc_s
            sq_v[pl.ds(t * 16, 16)] = acc_q

        col = lane * 16
        s_tot = jnp.zeros((16,), jnp.float32)
        q_tot = jnp.zeros((16,), jnp.float32)
        for l in range(16):
            s_tot = s_tot + plsc.load_gather(sums_v, [col + l])
            q_tot = q_tot + plsc.load_gather(sq_v, [col + l])
        mean = s_tot * (1.0 / HIDDEN)
        x = q_tot * (1.0 / HIDDEN) - mean * mean + LN_EPS
        iv = plsc.bitcast(x, jnp.int32)
        iv = 0x5F3759DF - lax.shift_right_logical(iv, 1)
        y = plsc.bitcast(iv, jnp.float32)
        xh = x * 0.5
        y = y * (1.5 - xh * y * y)
        y = y * (1.5 - xh * y * y)
        y = y * (1.5 - xh * y * y)
        a_v[pl.ds(0, CHUNK)] = y
        b_v[pl.ds(0, CHUNK)] = mean * y

        @plsc.parallel_loop(0, CHUNK // 4)
        def p3(g):
            t0 = g * 4
            ab = []
            for j in range(4):
                ti = jnp.full((16,), t0 + j, jnp.int32)
                ab.append((plsc.load_gather(a_v, [ti]),
                           plsc.load_gather(b_v, [ti])))

            @plsc.parallel_loop(0, HS, unroll=4)
            def apply(h):
                hs = pl.ds(h * 16, 16)
                gm = gam_v[hs]
                bt = bet_v[hs]
                for j in range(4):
                    a, b = ab[j]
                    outb[t0 + j, hs] = (rows[t0 + j, hs] * a - b) * gm + bt

    issue(0, rows0_v, sem_w0)
    issue(1, rows1_v, sem_w1)

    def pair(c, _):
        for k in (0, 1):
            rows, outb, sw, so = bufs[k]
            ci = 2 * c + k
            pltpu.make_async_copy(word_hbm.at[widx_v.at[ci]], rows, sw).wait()

            @pl.when(ci >= 2)
            def _():
                pltpu.make_async_copy(
                    outb, out_hbm.at[pl.ds(wid * SP_W, CHUNK)], so).wait()

            compute(ci, rows, outb)
            pltpu.async_copy(
                outb, out_hbm.at[pl.ds(ci * S_LEN + wid * SP_W, CHUNK)], so)

            @pl.when(ci + 2 < NCH)
            def _():
                issue(ci + 2, rows, sw)
        return 0

    lax.fori_loop(0, NCH // 2, pair, 0)

    pltpu.make_async_copy(outb0_v, out_hbm.at[pl.ds(0, CHUNK)], sem_o0).wait()
    pltpu.make_async_copy(outb1_v, out_hbm.at[pl.ds(0, CHUNK)], sem_o1).wait()


def kernel(input_ids, token_type_ids, word_emb, token_type_emb, pos_emb,
           ln_gamma, ln_beta):
    ids = input_ids.reshape(B_SZ, NW, SP_W).transpose(1, 0, 2)
    brow = (token_type_ids.reshape(B_SZ, NW, SP_W).transpose(1, 0, 2) * SP_W
            + jnp.arange(SP_W, dtype=jnp.int32)[None, None, :])
    out = _emb_layernorm(ids, brow, word_emb, token_type_emb,
                         pos_emb[:S_LEN], ln_gamma, ln_beta)
    return out.reshape(B_SZ, S_LEN, HIDDEN)


# speedup vs baseline: 1.1400x; 1.1400x over previous
import functools

import jax
import jax.numpy as jnp
from jax import lax
from jax.experimental import pallas as pl
from jax.experimental.pallas import tpu as pltpu
from jax.experimental.pallas import tpu_sc as plsc

VOCAB = 100000
HIDDEN = 768
S_LEN = 512
TT_VOCAB = 2
B_SZ = 64
N_TOK = B_SZ * S_LEN
LN_EPS = 1e-12

NW = 32
SP_W = S_LEN // NW
CHUNK = 16
NCH = B_SZ
HS = HIDDEN // 16
NBROWS = TT_VOCAB * SP_W

_mesh = plsc.VectorSubcoreMesh(core_axis_name="c", subcore_axis_name="s")
_cparams = pltpu.CompilerParams(needs_layout_passes=False)


@functools.partial(
    pl.kernel,
    mesh=_mesh,
    out_type=jax.ShapeDtypeStruct((N_TOK, HIDDEN), jnp.float32),
    compiler_params=_cparams,
    scratch_types=[
        pltpu.VMEM((NCH, CHUNK), jnp.int32),
        pltpu.VMEM((NCH, CHUNK), jnp.int32),
        pltpu.VMEM((CHUNK, HIDDEN), jnp.float32),
        pltpu.VMEM((CHUNK, HIDDEN), jnp.float32),
        pltpu.VMEM((CHUNK, HIDDEN), jnp.float32),
        pltpu.VMEM((CHUNK, HIDDEN), jnp.float32),
        pltpu.VMEM((NBROWS, HIDDEN), jnp.float32),
        pltpu.VMEM((TT_VOCAB, HIDDEN), jnp.float32),
        pltpu.VMEM((HIDDEN,), jnp.float32),
        pltpu.VMEM((HIDDEN,), jnp.float32),
        pltpu.VMEM((CHUNK * 16,), jnp.float32),
        pltpu.VMEM((CHUNK * 16,), jnp.float32),
        pltpu.VMEM((CHUNK,), jnp.float32),
        pltpu.VMEM((CHUNK,), jnp.float32),
        pltpu.SemaphoreType.DMA,
        pltpu.SemaphoreType.DMA,
        pltpu.SemaphoreType.DMA,
        pltpu.SemaphoreType.DMA,
    ],
)
def _emb_layernorm(ids_hbm, brow_hbm, word_hbm, tt_hbm, pos_hbm, gam_hbm,
                   bet_hbm, out_hbm, widx_v, brow_v, rows0_v, rows1_v,
                   outb0_v, outb1_v, ttb_v, ttbuf_v, gam_v, bet_v,
                   sums_v, sq_v, a_v, b_v, sem_w0, sem_w1, sem_o0, sem_o1):
    wid = lax.axis_index("s") * 2 + lax.axis_index("c")
    pltpu.sync_copy(ids_hbm.at[wid], widx_v)
    pltpu.sync_copy(brow_hbm.at[wid], brow_v)
    pltpu.sync_copy(gam_hbm, gam_v)
    pltpu.sync_copy(bet_hbm, bet_v)
    pltpu.sync_copy(tt_hbm, ttbuf_v)
    pltpu.sync_copy(pos_hbm.at[pl.ds(wid * SP_W, SP_W)],
                    ttb_v.at[pl.ds(0, SP_W)])
    pltpu.sync_copy(pos_hbm.at[pl.ds(wid * SP_W, SP_W)],
                    ttb_v.at[pl.ds(SP_W, SP_W)])

    @plsc.parallel_loop(0, NBROWS * HS, unroll=8)
    def build(i):
        r = i // HS
        hs = pl.ds((i % HS) * 16, 16)
        ttb_v[r, hs] = ttb_v[r, hs] + ttbuf_v[r // SP_W, hs]

    lane = lax.iota(jnp.int32, 16)
    bufs = ((rows0_v, outb0_v, sem_w0, sem_o0),
            (rows1_v, outb1_v, sem_w1, sem_o1))

    def issue(ci, rows, sw):
        pltpu.async_copy(word_hbm.at[widx_v.at[ci]], rows, sw)

    def compute(ci, rows, outb):
        @plsc.parallel_loop(0, CHUNK)
        def p1(t):
            br = plsc.load_gather(brow_v, [jnp.full((16,), ci, jnp.int32),
                                           jnp.full((16,), t, jnp.int32)])
            z = jnp.zeros((16,), jnp.float32)

            @plsc.parallel_loop(0, HS, unroll=8, carry=(z, z))
            def accs(h, acc):
                acc_s, acc_q = acc
                hs = pl.ds(h * 16, 16)
                bslice = plsc.load_gather(ttb_v, [br, h * 16 + lane])
                v = rows[t, hs] + bslice
                rows[t, hs] = v
                return (acc_s + v, acc_q + v * v)

            acc_s, acc_q = accs
            sums_v[pl.ds(t * 16, 16)] = acc_s
            sq_v[pl.ds(t * 16, 16)] = acc_q

        col = lane * 16
        s_tot = jnp.zeros((16,), jnp.float32)
        q_tot = jnp.zeros((16,), jnp.float32)
        for l in range(16):
            s_tot = s_tot + plsc.load_gather(sums_v, [col + l])
            q_tot = q_tot + plsc.load_gather(sq_v, [col + l])
        mean = s_tot * (1.0 / HIDDEN)
        x = q_tot * (1.0 / HIDDEN) - mean * mean + LN_EPS
        iv = plsc.bitcast(x, jnp.int32)
        iv = 0x5F3759DF - lax.shift_right_logical(iv, 1)
        y = plsc.bitcast(iv, jnp.float32)
        xh = x * 0.5
        y = y * (1.5 - xh * y * y)
        y = y * (1.5 - xh * y * y)
        y = y * (1.5 - xh * y * y)
        a_v[pl.ds(0, CHUNK)] = y
        b_v[pl.ds(0, CHUNK)] = mean * y

        @plsc.parallel_loop(0, CHUNK // 8)
        def p3(g):
            t0 = g * 8
            ab = []
            for j in range(8):
                ti = jnp.full((16,), t0 + j, jnp.int32)
                ab.append((plsc.load_gather(a_v, [ti]),
                           plsc.load_gather(b_v, [ti])))

            @plsc.parallel_loop(0, HS, unroll=2)
            def apply(h):
                hs = pl.ds(h * 16, 16)
                gm = gam_v[hs]
                bt = bet_v[hs]
                for j in range(8):
                    a, b = ab[j]
                    outb[t0 + j, hs] = (rows[t0 + j, hs] * a - b) * gm + bt

    issue(0, rows0_v, sem_w0)
    issue(1, rows1_v, sem_w1)

    def pair(c, _):
        for k in (0, 1):
            rows, outb, sw, so = bufs[k]
            ci = 2 * c + k
            pltpu.make_async_copy(word_hbm.at[widx_v.at[ci]], rows, sw).wait()

            @pl.when(ci >= 2)
            def _():
                pltpu.make_async_copy(
                    outb, out_hbm.at[pl.ds(wid * SP_W, CHUNK)], so).wait()

            compute(ci, rows, outb)
            pltpu.async_copy(
                outb, out_hbm.at[pl.ds(ci * S_LEN + wid * SP_W, CHUNK)], so)

            @pl.when(ci + 2 < NCH)
            def _():
                issue(ci + 2, rows, sw)
        return 0

    lax.fori_loop(0, NCH // 2, pair, 0)

    pltpu.make_async_copy(outb0_v, out_hbm.at[pl.ds(0, CHUNK)], sem_o0).wait()
    pltpu.make_async_copy(outb1_v, out_hbm.at[pl.ds(0, CHUNK)], sem_o1).wait()


def kernel(input_ids, token_type_ids, word_emb, token_type_emb, pos_emb,
           ln_gamma, ln_beta):
    ids = input_ids.reshape(B_SZ, NW, SP_W).transpose(1, 0, 2)
    brow = (token_type_ids.reshape(B_SZ, NW, SP_W).transpose(1, 0, 2) * SP_W
            + jnp.arange(SP_W, dtype=jnp.int32)[None, None, :])
    out = _emb_layernorm(ids, brow, word_emb, token_type_emb,
                         pos_emb[:S_LEN], ln_gamma, ln_beta)
    return out.reshape(B_SZ, S_LEN, HIDDEN)
